# baseline (speedup 1.0000x reference)
import functools

import jax
import jax.numpy as jnp
from jax import lax
from jax.experimental import pallas as pl
from jax.experimental.pallas import tpu as pltpu
from jax.experimental.pallas import tpu_sc as plsc

B = 16384
V = 1000000
D = 64
NNEG = 20
L = 16
DC = D // L
NC = 2
NS = 16
NW = NC * NS
TPW = B // NW
T = 16
NCHUNK = TPW // T
NC2 = NCHUNK // 2
NR = T * NNEG
NG = NR // L

_DNUMS = lax.GatherDimensionNumbers(
    offset_dims=(), collapsed_slice_dims=(0,), start_index_map=(0,))


def _take(v, idx):
    return lax.gather(v, idx[:, None], _DNUMS, (1,),
                      mode=lax.GatherScatterMode.PROMISE_IN_BOUNDS)


def _assemble(partials, perms, masks):
    vecs = list(partials)
    for lvl in range(4):
        perm, m = perms[lvl], masks[lvl]
        nxt = []
        for j in range(0, len(vecs), 2):
            a, b = vecs[j], vecs[j + 1]
            ta = a + _take(a, perm)
            tb = b + _take(b, perm)
            nxt.append(jnp.where(m, ta, tb))
        vecs = nxt
    return vecs[0]


def _sc_body(center_hbm, context_hbm, neg_hbm, cw_hbm, xw_hbm,
             pos_hbm, negout_hbm,
             cidx, xidx, nidx,
             crowA, xrowA, nrow0A, nrow1A, nrow2A,
             crowB, xrowB, nrow0B, nrow1B, nrow2B,
             posb, negb, semA, semB):
    cid = lax.axis_index("c")
    sid = lax.axis_index("s")
    wid = sid * NC + cid
    base = wid * TPW

    bufsA = (crowA, xrowA, nrow0A, nrow1A, nrow2A)
    bufsB = (crowB, xrowB, nrow0B, nrow1B, nrow2B)

    lane = lax.iota(jnp.int32, L)
    perms = [lane ^ s for s in (1, 2, 4, 8)]
    masks = [(lane & s) == 0 for s in (1, 2, 4, 8)]

    pltpu.sync_copy(center_hbm.at[pl.ds(base, TPW)], cidx)
    pltpu.sync_copy(context_hbm.at[pl.ds(base, TPW)], xidx)
    pltpu.sync_copy(neg_hbm.at[pl.ds(base * NNEG, TPW * NNEG)], nidx)

    def issue(c, bufs, sem):
        crow, xrow, n0, n1, n2 = bufs
        o = c * T
        no = c * NR
        pltpu.async_copy(cw_hbm.at[cidx.at[pl.ds(o, T)]], crow, sem)
        pltpu.async_copy(xw_hbm.at[xidx.at[pl.ds(o, T)]], xrow, sem)
        pltpu.async_copy(xw_hbm.at[nidx.at[pl.ds(no, 128)]], n0, sem)
        pltpu.async_copy(xw_hbm.at[nidx.at[pl.ds(no + 128, 128)]], n1, sem)
        pltpu.async_copy(xw_hbm.at[nidx.at[pl.ds(no + 256, 64)]], n2, sem)

    def drain(bufs, sem):
        for d in bufs:
            n = d.shape[0]
            pltpu.make_async_copy(cw_hbm.at[pl.ds(0, n)], d, sem).wait()

    def compute(c, bufs):
        crow, xrow, n0, n1, n2 = bufs
        nrow_refs = ((n0, 0), (n1, 128), (n2, 256))

        def _nrow(r):
            for ref, off in reversed(nrow_refs):
                if r >= off:
                    return ref, r - off
            raise AssertionError

        nbase = c * NR
        for g in range(NG):
            r0 = g * L
            cvec = {}
            partials = []
            for i in range(L):
                r = r0 + i
                t = r // NNEG
                if t not in cvec:
                    cvec[t] = [crow[t, pl.ds(dc * L, L)] for dc in range(DC)]
                nref, rr = _nrow(r)
                acc = nref[rr, pl.ds(0, L)] * cvec[t][0]
                for dc in range(1, DC):
                    acc = acc + nref[rr, pl.ds(dc * L, L)] * cvec[t][dc]
                partials.append(acc)
            negb[pl.ds(nbase + r0, L)] = _assemble(partials, perms, masks)

        partials = []
        for t in range(T):
            acc = crow[t, pl.ds(0, L)] * xrow[t, pl.ds(0, L)]
            for dc in range(1, DC):
                acc = acc + crow[t, pl.ds(dc * L, L)] * xrow[t, pl.ds(dc * L, L)]
            partials.append(acc)
        posb[pl.ds(c * T, T)] = _assemble(partials, perms, masks)

    issue(0, bufsA, semA)

    def step(c2, carry):
        c0 = c2 * 2
        c1 = c0 + 1
        issue(c1, bufsB, semB)
        drain(bufsA, semA)
        compute(c0, bufsA)
        cnext = lax.rem(c0 + 2, NCHUNK)
        issue(cnext, bufsA, semA)
        drain(bufsB, semB)
        compute(c1, bufsB)
        return carry

    lax.fori_loop(0, NC2, step, 0)
    drain(bufsA, semA)

    pltpu.sync_copy(posb, pos_hbm.at[pl.ds(base, TPW)])
    pltpu.sync_copy(negb, negout_hbm.at[pl.ds(base * NNEG, TPW * NNEG)])


_sc_kernel = functools.partial(
    pl.kernel,
    out_type=[
        jax.ShapeDtypeStruct((B,), jnp.float32),
        jax.ShapeDtypeStruct((B * NNEG,), jnp.float32),
    ],
    mesh=plsc.VectorSubcoreMesh(core_axis_name="c", subcore_axis_name="s"),
    compiler_params=pltpu.CompilerParams(use_tc_tiling_on_sc=False),
    scratch_types=[
        pltpu.VMEM((TPW,), jnp.int32),
        pltpu.VMEM((TPW,), jnp.int32),
        pltpu.VMEM((TPW * NNEG,), jnp.int32),
        pltpu.VMEM((T, D), jnp.float32),
        pltpu.VMEM((T, D), jnp.float32),
        pltpu.VMEM((128, D), jnp.float32),
        pltpu.VMEM((128, D), jnp.float32),
        pltpu.VMEM((64, D), jnp.float32),
        pltpu.VMEM((T, D), jnp.float32),
        pltpu.VMEM((T, D), jnp.float32),
        pltpu.VMEM((128, D), jnp.float32),
        pltpu.VMEM((128, D), jnp.float32),
        pltpu.VMEM((64, D), jnp.float32),
        pltpu.VMEM((TPW,), jnp.float32),
        pltpu.VMEM((TPW * NNEG,), jnp.float32),
        pltpu.SemaphoreType.DMA,
        pltpu.SemaphoreType.DMA,
    ],
)(_sc_body)


def kernel(center, context, negatives, center_weight, context_weight):
    one = 1.0 + 0.0 * center[0].astype(jnp.float32)
    center_weight = center_weight * one
    context_weight = context_weight * one
    negflat = negatives.reshape(-1).astype(jnp.int32)
    pos, negf = _sc_kernel(
        center.astype(jnp.int32),
        context.astype(jnp.int32),
        negflat,
        center_weight,
        context_weight,
    )
    return pos, negf.reshape(B, NNEG)


# speedup vs baseline: 1.5990x; 1.5990x over previous
import functools

import jax
import jax.numpy as jnp
from jax import lax
from jax.experimental import pallas as pl
from jax.experimental.pallas import tpu as pltpu
from jax.experimental.pallas import tpu_sc as plsc

B = 16384
V = 1000000
VH = V // 2
D = 64
NNEG = 20
L = 16
DC = D // L
NC = 2
NS = 16
NW = NC * NS
TPW = B // NW
T = 16
NCHUNK = TPW // T
NC2 = NCHUNK // 2
NR = T * NNEG
NG = NR // L
NID = TPW * NNEG
SG = 80

_DNUMS = lax.GatherDimensionNumbers(
    offset_dims=(), collapsed_slice_dims=(0,), start_index_map=(0,))


def _take(v, idx):
    return lax.gather(v, idx[:, None], _DNUMS, (1,),
                      mode=lax.GatherScatterMode.PROMISE_IN_BOUNDS)


def _bcast(v, i):
    return _take(v, jnp.full((L,), i, jnp.int32))


class _TreeSum:
    def __init__(self, perms, masks):
        self.perms, self.masks = perms, masks
        self.stack = []

    def push(self, vec):
        lvl = 0
        while self.stack and self.stack[-1][0] == lvl:
            _, a = self.stack.pop()
            ta = a + _take(a, self.perms[lvl])
            tb = vec + _take(vec, self.perms[lvl])
            vec = jnp.where(self.masks[lvl], ta, tb)
            lvl += 1
        self.stack.append((lvl, vec))

    def finish(self):
        (_, v), = self.stack
        self.stack = []
        return v


def _sc_body(center_hbm, context_hbm, neg_hbm, cw_hbm, xw_hbm,
             pos_hbm, negout_hbm,
             cidx, xidx, nidx, hidxA, hidxB,
             crowA, xrowA, nrowA, crowB, xrowB, nrowB,
             posb, negb, semA, semB):
    cid = lax.axis_index("c")
    sid = lax.axis_index("s")
    wid = sid * NC + cid
    base = wid * TPW

    bufsA = (crowA, xrowA, nrowA)
    bufsB = (crowB, xrowB, nrowB)

    lane = lax.iota(jnp.int32, L)
    perms = [lane ^ s for s in (1, 2, 4, 8)]
    masks = [(lane & s) == 0 for s in (1, 2, 4, 8)]

    pltpu.sync_copy(center_hbm.at[pl.ds(base, TPW)], cidx)
    pltpu.sync_copy(context_hbm.at[pl.ds(base, TPW)], xidx)
    pltpu.sync_copy(neg_hbm.at[pl.ds(base * NNEG, NID)], nidx)

    def issue(c, bufs, hidx, sem):
        crow, xrow, nrow = bufs
        o = c * T
        no = c * NR
        hidx[pl.ds(0, T)] = lax.shift_right_logical(cidx[pl.ds(o, T)], 1)
        hidx[pl.ds(T, T)] = lax.shift_right_logical(xidx[pl.ds(o, T)], 1)
        for j in range(NR // L):
            hidx[pl.ds(2 * T + j * L, L)] = lax.shift_right_logical(
                nidx[pl.ds(no + j * L, L)], 1)
        pltpu.async_copy(cw_hbm.at[hidx.at[pl.ds(0, T)]], crow, sem)
        pltpu.async_copy(xw_hbm.at[hidx.at[pl.ds(T, T)]], xrow, sem)
        for k in range(4):
            pltpu.async_copy(xw_hbm.at[hidx.at[pl.ds(2 * T + SG * k, SG)]],
                             nrow.at[pl.ds(SG * k, SG)], sem)

    def drain(bufs, sem):
        crow, xrow, nrow = bufs
        for d, n in ((crow, T), (xrow, T), (nrow, SG), (nrow, SG),
                     (nrow, SG), (nrow, SG)):
            pltpu.make_async_copy(cw_hbm.at[pl.ds(0, n)],
                                  d.at[pl.ds(0, n)], sem).wait()

    def _halfrow(ref, r, pm):
        return [jnp.where(pm, ref[r, pl.ds(D + dc * L, L)],
                          ref[r, pl.ds(dc * L, L)]) for dc in range(DC)]

    def compute(c, bufs):
        crow, xrow, nrow = bufs
        cparv = cidx[pl.ds(c * T, T)] & 1
        xparv = xidx[pl.ds(c * T, T)] & 1
        nbase = c * NR

        def sg(sgi, carry):
            t4 = sgi * 4
            rb = sgi * SG
            for gg in range(SG // L):
                r0 = gg * L
                nparv = nidx[pl.ds(nbase + rb + r0, L)] & 1
                cvec = {}
                tree = _TreeSum(perms, masks)
                for i in range(L):
                    rs = r0 + i
                    tloc = rs // NNEG
                    t = t4 + tloc
                    if tloc not in cvec:
                        cvec[tloc] = _halfrow(crow, t, _bcast(cparv, t) == 1)
                    nv = _halfrow(nrow, rb + rs, _bcast(nparv, i) == 1)
                    acc = nv[0] * cvec[tloc][0]
                    for dc in range(1, DC):
                        acc = acc + nv[dc] * cvec[tloc][dc]
                    tree.push(acc)
                negb[pl.ds(nbase + rb + r0, L)] = tree.finish()
            return carry

        lax.fori_loop(0, NR // SG, sg, 0)

        tree = _TreeSum(perms, masks)
        for t in range(T):
            cv = _halfrow(crow, t, _bcast(cparv, t) == 1)
            xv = _halfrow(xrow, t, _bcast(xparv, t) == 1)
            acc = cv[0] * xv[0]
            for dc in range(1, DC):
                acc = acc + cv[dc] * xv[dc]
            tree.push(acc)
        posb[pl.ds(c * T, T)] = tree.finish()

    issue(0, bufsA, hidxA, semA)

    def step(c2, carry):
        c0 = c2 * 2
        c1 = c0 + 1
        issue(c1, bufsB, hidxB, semB)
        drain(bufsA, semA)
        compute(c0, bufsA)
        cnext = lax.rem(c0 + 2, NCHUNK)
        issue(cnext, bufsA, hidxA, semA)
        drain(bufsB, semB)
        compute(c1, bufsB)
        return carry

    lax.fori_loop(0, NC2, step, 0)
    drain(bufsA, semA)

    pltpu.sync_copy(posb, pos_hbm.at[pl.ds(base, TPW)])
    pltpu.sync_copy(negb, negout_hbm.at[pl.ds(base * NNEG, NID)])


_sc_kernel = functools.partial(
    pl.kernel,
    out_type=[
        jax.ShapeDtypeStruct((B,), jnp.float32),
        jax.ShapeDtypeStruct((B * NNEG,), jnp.float32),
    ],
    mesh=plsc.VectorSubcoreMesh(core_axis_name="c", subcore_axis_name="s"),
    compiler_params=pltpu.CompilerParams(needs_layout_passes=False),
    scratch_types=[
        pltpu.VMEM((TPW,), jnp.int32),
        pltpu.VMEM((TPW,), jnp.int32),
        pltpu.VMEM((NID,), jnp.int32),
        pltpu.VMEM((2 * T + NR,), jnp.int32),
        pltpu.VMEM((2 * T + NR,), jnp.int32),
        pltpu.VMEM((T, 2 * D), jnp.float32),
        pltpu.VMEM((T, 2 * D), jnp.float32),
        pltpu.VMEM((NR, 2 * D), jnp.float32),
        pltpu.VMEM((T, 2 * D), jnp.float32),
        pltpu.VMEM((T, 2 * D), jnp.float32),
        pltpu.VMEM((NR, 2 * D), jnp.float32),
        pltpu.VMEM((TPW,), jnp.float32),
        pltpu.VMEM((NID,), jnp.float32),
        pltpu.SemaphoreType.DMA,
        pltpu.SemaphoreType.DMA,
    ],
)(_sc_body)


def kernel(center, context, negatives, center_weight, context_weight):
    negflat = negatives.reshape(-1).astype(jnp.int32)
    pos, negf = _sc_kernel(
        center.astype(jnp.int32),
        context.astype(jnp.int32),
        negflat,
        center_weight.reshape(VH, 2 * D),
        context_weight.reshape(VH, 2 * D),
    )
    return pos, negf.reshape(B, NNEG)


# speedup vs baseline: 2.1221x; 1.3271x over previous
import functools

import jax
import jax.numpy as jnp
from jax import lax
from jax.experimental import pallas as pl
from jax.experimental.pallas import tpu as pltpu
from jax.experimental.pallas import tpu_sc as plsc

B = 16384
V = 1000000
TB = 2048
NBLK = -(-V // TB)
VROWS = NBLK * (TB // 2)
D = 64
NNEG = 20
L = 16
DC = D // L
NC = 2
NS = 16
NW = NC * NS
TPW = B // NW
T = 16
NCHUNK = TPW // T
NC2 = NCHUNK // 2
NR = T * NNEG
NG = NR // L
NID = TPW * NNEG
SG = 80

_DNUMS = lax.GatherDimensionNumbers(
    offset_dims=(), collapsed_slice_dims=(0,), start_index_map=(0,))


def _take(v, idx):
    return lax.gather(v, idx[:, None], _DNUMS, (1,),
                      mode=lax.GatherScatterMode.PROMISE_IN_BOUNDS)


def _bcast(v, i):
    return _take(v, jnp.full((L,), i, jnp.int32))


def _pair_body(in_ref, out_ref):
    blk = in_ref[...]
    c = jnp.concatenate([blk[:, :TB // 2], blk[:, TB // 2:]], axis=0)
    out_ref[...] = c.T


_pair_transpose = pl.pallas_call(
    _pair_body,
    grid=(NBLK,),
    in_specs=[pl.BlockSpec((D, TB), lambda i: (0, i))],
    out_specs=pl.BlockSpec((TB // 2, 2 * D), lambda i: (i, 0)),
    out_shape=jax.ShapeDtypeStruct((VROWS, 2 * D), jnp.float32),
)


class _TreeSum:
    def __init__(self, perms, masks):
        self.perms, self.masks = perms, masks
        self.stack = []

    def push(self, vec):
        lvl = 0
        while self.stack and self.stack[-1][0] == lvl:
            _, a = self.stack.pop()
            ta = a + _take(a, self.perms[lvl])
            tb = vec + _take(vec, self.perms[lvl])
            vec = jnp.where(self.masks[lvl], ta, tb)
            lvl += 1
        self.stack.append((lvl, vec))

    def finish(self):
        (_, v), = self.stack
        self.stack = []
        return v


def _sc_body(center_hbm, context_hbm, neg_hbm, cw_hbm, xw_hbm,
             pos_hbm, negout_hbm,
             cidx, xidx, nidx, hidxA, hidxB,
             crowA, xrowA, nrowA, crowB, xrowB, nrowB,
             posb, negb, semA, semB):
    cid = lax.axis_index("c")
    sid = lax.axis_index("s")
    wid = sid * NC + cid
    base = wid * TPW

    bufsA = (crowA, xrowA, nrowA)
    bufsB = (crowB, xrowB, nrowB)

    lane = lax.iota(jnp.int32, L)
    perms = [lane ^ s for s in (1, 2, 4, 8)]
    masks = [(lane & s) == 0 for s in (1, 2, 4, 8)]

    pltpu.sync_copy(center_hbm.at[pl.ds(base, TPW)], cidx)
    pltpu.sync_copy(context_hbm.at[pl.ds(base, TPW)], xidx)
    pltpu.sync_copy(neg_hbm.at[pl.ds(base * NNEG, NID)], nidx)

    def issue(c, bufs, hidx, sem):
        crow, xrow, nrow = bufs
        o = c * T
        no = c * NR
        def _pairrow(vv):
            return lax.shift_left(
                lax.shift_right_logical(vv, 11), 10) | (vv & 1023)

        hidx[pl.ds(0, T)] = _pairrow(cidx[pl.ds(o, T)])
        hidx[pl.ds(T, T)] = _pairrow(xidx[pl.ds(o, T)])
        for j in range(NR // L):
            hidx[pl.ds(2 * T + j * L, L)] = _pairrow(nidx[pl.ds(no + j * L, L)])
        pltpu.async_copy(cw_hbm.at[hidx.at[pl.ds(0, T)]], crow, sem)
        pltpu.async_copy(xw_hbm.at[hidx.at[pl.ds(T, T)]], xrow, sem)
        for k in range(4):
            pltpu.async_copy(xw_hbm.at[hidx.at[pl.ds(2 * T + SG * k, SG)]],
                             nrow.at[pl.ds(SG * k, SG)], sem)

    def drain(bufs, sem):
        crow, xrow, nrow = bufs
        for d, n in ((crow, T), (xrow, T), (nrow, SG), (nrow, SG),
                     (nrow, SG), (nrow, SG)):
            pltpu.make_async_copy(cw_hbm.at[pl.ds(0, n)],
                                  d.at[pl.ds(0, n)], sem).wait()

    def _halfrow(ref, r, pm):
        return [jnp.where(pm, ref[r, pl.ds(D + dc * L, L)],
                          ref[r, pl.ds(dc * L, L)]) for dc in range(DC)]

    def compute(c, bufs):
        crow, xrow, nrow = bufs
        cparv = lax.shift_right_logical(cidx[pl.ds(c * T, T)], 10) & 1
        xparv = lax.shift_right_logical(xidx[pl.ds(c * T, T)], 10) & 1
        nbase = c * NR

        def sg(sgi, carry):
            t4 = sgi * 4
            rb = sgi * SG
            for gg in range(SG // L):
                r0 = gg * L
                nparv = lax.shift_right_logical(nidx[pl.ds(nbase + rb + r0, L)], 10) & 1
                cvec = {}
                tree = _TreeSum(perms, masks)
                for i in range(L):
                    rs = r0 + i
                    tloc = rs // NNEG
                    t = t4 + tloc
                    if tloc not in cvec:
                        cvec[tloc] = _halfrow(crow, t, _bcast(cparv, t) == 1)
                    nv = _halfrow(nrow, rb + rs, _bcast(nparv, i) == 1)
                    acc = nv[0] * cvec[tloc][0]
                    for dc in range(1, DC):
                        acc = acc + nv[dc] * cvec[tloc][dc]
                    tree.push(acc)
                negb[pl.ds(nbase + rb + r0, L)] = tree.finish()
            return carry

        lax.fori_loop(0, NR // SG, sg, 0)

        tree = _TreeSum(perms, masks)
        for t in range(T):
            cv = _halfrow(crow, t, _bcast(cparv, t) == 1)
            xv = _halfrow(xrow, t, _bcast(xparv, t) == 1)
            acc = cv[0] * xv[0]
            for dc in range(1, DC):
                acc = acc + cv[dc] * xv[dc]
            tree.push(acc)
        posb[pl.ds(c * T, T)] = tree.finish()

    issue(0, bufsA, hidxA, semA)

    def step(c2, carry):
        c0 = c2 * 2
        c1 = c0 + 1
        issue(c1, bufsB, hidxB, semB)
        drain(bufsA, semA)
        compute(c0, bufsA)
        cnext = lax.rem(c0 + 2, NCHUNK)
        issue(cnext, bufsA, hidxA, semA)
        drain(bufsB, semB)
        compute(c1, bufsB)
        return carry

    lax.fori_loop(0, NC2, step, 0)
    drain(bufsA, semA)

    pltpu.sync_copy(posb, pos_hbm.at[pl.ds(base, TPW)])
    pltpu.sync_copy(negb, negout_hbm.at[pl.ds(base * NNEG, NID)])


_sc_kernel = functools.partial(
    pl.kernel,
    out_type=[
        jax.ShapeDtypeStruct((B,), jnp.float32),
        jax.ShapeDtypeStruct((B * NNEG,), jnp.float32),
    ],
    mesh=plsc.VectorSubcoreMesh(core_axis_name="c", subcore_axis_name="s"),
    compiler_params=pltpu.CompilerParams(needs_layout_passes=False),
    scratch_types=[
        pltpu.VMEM((TPW,), jnp.int32),
        pltpu.VMEM((TPW,), jnp.int32),
        pltpu.VMEM((NID,), jnp.int32),
        pltpu.VMEM((2 * T + NR,), jnp.int32),
        pltpu.VMEM((2 * T + NR,), jnp.int32),
        pltpu.VMEM((T, 2 * D), jnp.float32),
        pltpu.VMEM((T, 2 * D), jnp.float32),
        pltpu.VMEM((NR, 2 * D), jnp.float32),
        pltpu.VMEM((T, 2 * D), jnp.float32),
        pltpu.VMEM((T, 2 * D), jnp.float32),
        pltpu.VMEM((NR, 2 * D), jnp.float32),
        pltpu.VMEM((TPW,), jnp.float32),
        pltpu.VMEM((NID,), jnp.float32),
        pltpu.SemaphoreType.DMA,
        pltpu.SemaphoreType.DMA,
    ],
)(_sc_body)


def kernel(center, context, negatives, center_weight, context_weight):
    negflat = negatives.reshape(-1).astype(jnp.int32)
    pos, negf = _sc_kernel(
        center.astype(jnp.int32),
        context.astype(jnp.int32),
        negflat,
        _pair_transpose(center_weight.T),
        _pair_transpose(context_weight.T),
    )
    return pos, negf.reshape(B, NNEG)


# speedup vs baseline: 2.7844x; 1.3121x over previous
import functools

import jax
import jax.numpy as jnp
from jax import lax
from jax.experimental import pallas as pl
from jax.experimental.pallas import tpu as pltpu
from jax.experimental.pallas import tpu_sc as plsc

B = 16384
V = 1000000
TB = 4096
TBL = TB.bit_length() - 1
NBLK = -(-V // TB)
VROWS = NBLK * (TB // 2)
D = 64
NNEG = 20
L = 16
DC = D // L
NC = 2
NS = 16
NW = NC * NS
TPW = B // NW
T = 16
NCHUNK = TPW // T
NC2 = NCHUNK // 2
NR = T * NNEG
NG = NR // L
NID = TPW * NNEG
SG = 80

_DNUMS = lax.GatherDimensionNumbers(
    offset_dims=(), collapsed_slice_dims=(0,), start_index_map=(0,))


def _take(v, idx):
    return lax.gather(v, idx[:, None], _DNUMS, (1,),
                      mode=lax.GatherScatterMode.PROMISE_IN_BOUNDS)


def _bcast(v, i):
    return _take(v, jnp.full((L,), i, jnp.int32))


def _pair_body(in_ref, out_ref):
    blk = in_ref[...]
    c = jnp.concatenate([blk[:, :TB // 2], blk[:, TB // 2:]], axis=0)
    out_ref[...] = c.T


_pair_transpose = pl.pallas_call(
    _pair_body,
    grid=(NBLK,),
    in_specs=[pl.BlockSpec((D, TB), lambda i: (0, i))],
    out_specs=pl.BlockSpec((TB // 2, 2 * D), lambda i: (i, 0)),
    out_shape=jax.ShapeDtypeStruct((VROWS, 2 * D), jnp.float32),
)


class _TreeSum:
    def __init__(self, perms, masks):
        self.perms, self.masks = perms, masks
        self.stack = []

    def push(self, vec):
        lvl = 0
        while self.stack and self.stack[-1][0] == lvl:
            _, a = self.stack.pop()
            ta = a + _take(a, self.perms[lvl])
            tb = vec + _take(vec, self.perms[lvl])
            vec = jnp.where(self.masks[lvl], ta, tb)
            lvl += 1
        self.stack.append((lvl, vec))

    def finish(self):
        (_, v), = self.stack
        self.stack = []
        return v


def _sc_body(center_hbm, context_hbm, neg_hbm, cw_hbm, xw_hbm,
             pos_hbm, negout_hbm,
             cidx, xidx, nidx, hidxA, hidxB,
             crowA, xrowA, nrowA, crowB, xrowB, nrowB,
             posb, negb, semA, semB):
    cid = lax.axis_index("c")
    sid = lax.axis_index("s")
    wid = sid * NC + cid
    base = wid * TPW

    bufsA = (crowA, xrowA, nrowA)
    bufsB = (crowB, xrowB, nrowB)

    lane = lax.iota(jnp.int32, L)
    perms = [lane ^ s for s in (1, 2, 4, 8)]
    masks = [(lane & s) == 0 for s in (1, 2, 4, 8)]

    pltpu.sync_copy(center_hbm.at[pl.ds(base, TPW)], cidx)
    pltpu.sync_copy(context_hbm.at[pl.ds(base, TPW)], xidx)
    pltpu.sync_copy(neg_hbm.at[pl.ds(base * NNEG, NID)], nidx)

    def issue(c, bufs, hidx, sem):
        crow, xrow, nrow = bufs
        o = c * T
        no = c * NR
        def _pairrow(vv):
            return lax.shift_left(
                lax.shift_right_logical(vv, TBL),
                TBL - 1) | (vv & (TB // 2 - 1))

        hidx[pl.ds(0, T)] = _pairrow(cidx[pl.ds(o, T)])
        hidx[pl.ds(T, T)] = _pairrow(xidx[pl.ds(o, T)])
        for j in range(NR // L):
            hidx[pl.ds(2 * T + j * L, L)] = _pairrow(nidx[pl.ds(no + j * L, L)])
        pltpu.async_copy(cw_hbm.at[hidx.at[pl.ds(0, T)]], crow, sem)
        pltpu.async_copy(xw_hbm.at[hidx.at[pl.ds(T, T)]], xrow, sem)
        for k in range(4):
            pltpu.async_copy(xw_hbm.at[hidx.at[pl.ds(2 * T + SG * k, SG)]],
                             nrow.at[pl.ds(SG * k, SG)], sem)

    def drain(bufs, sem):
        crow, xrow, nrow = bufs
        for d, n in ((crow, T), (xrow, T), (nrow, SG), (nrow, SG),
                     (nrow, SG), (nrow, SG)):
            pltpu.make_async_copy(cw_hbm.at[pl.ds(0, n)],
                                  d.at[pl.ds(0, n)], sem).wait()

    def _halfrow(ref, r, pm):
        return [jnp.where(pm, ref[r, pl.ds(D + dc * L, L)],
                          ref[r, pl.ds(dc * L, L)]) for dc in range(DC)]

    def compute(c, bufs):
        crow, xrow, nrow = bufs
        cparv = lax.shift_right_logical(cidx[pl.ds(c * T, T)], TBL - 1) & 1
        xparv = lax.shift_right_logical(xidx[pl.ds(c * T, T)], TBL - 1) & 1
        nbase = c * NR

        def sg(sgi, carry):
            t4 = sgi * 4
            rb = sgi * SG
            for gg in range(SG // L):
                r0 = gg * L
                nparv = lax.shift_right_logical(
                    nidx[pl.ds(nbase + rb + r0, L)], TBL - 1) & 1
                cvec = {}
                tree = _TreeSum(perms, masks)
                for i in range(L):
                    rs = r0 + i
                    tloc = rs // NNEG
                    t = t4 + tloc
                    if tloc not in cvec:
                        cvec[tloc] = _halfrow(crow, t, _bcast(cparv, t) == 1)
                    nv = _halfrow(nrow, rb + rs, _bcast(nparv, i) == 1)
                    acc = nv[0] * cvec[tloc][0]
                    for dc in range(1, DC):
                        acc = acc + nv[dc] * cvec[tloc][dc]
                    tree.push(acc)
                negb[pl.ds(nbase + rb + r0, L)] = tree.finish()
            return carry

        lax.fori_loop(0, NR // SG, sg, 0)

        tree = _TreeSum(perms, masks)
        for t in range(T):
            cv = _halfrow(crow, t, _bcast(cparv, t) == 1)
            xv = _halfrow(xrow, t, _bcast(xparv, t) == 1)
            acc = cv[0] * xv[0]
            for dc in range(1, DC):
                acc = acc + cv[dc] * xv[dc]
            tree.push(acc)
        posb[pl.ds(c * T, T)] = tree.finish()

    issue(0, bufsA, hidxA, semA)

    def step(c2, carry):
        c0 = c2 * 2
        c1 = c0 + 1
        issue(c1, bufsB, hidxB, semB)
        drain(bufsA, semA)
        compute(c0, bufsA)
        cnext = lax.rem(c0 + 2, NCHUNK)
        issue(cnext, bufsA, hidxA, semA)
        drain(bufsB, semB)
        compute(c1, bufsB)
        return carry

    lax.fori_loop(0, NC2, step, 0)
    drain(bufsA, semA)

    pltpu.sync_copy(posb, pos_hbm.at[pl.ds(base, TPW)])
    pltpu.sync_copy(negb, negout_hbm.at[pl.ds(base * NNEG, NID)])


_sc_kernel = functools.partial(
    pl.kernel,
    out_type=[
        jax.ShapeDtypeStruct((B,), jnp.float32),
        jax.ShapeDtypeStruct((B * NNEG,), jnp.float32),
    ],
    mesh=plsc.VectorSubcoreMesh(core_axis_name="c", subcore_axis_name="s"),
    compiler_params=pltpu.CompilerParams(needs_layout_passes=False),
    scratch_types=[
        pltpu.VMEM((TPW,), jnp.int32),
        pltpu.VMEM((TPW,), jnp.int32),
        pltpu.VMEM((NID,), jnp.int32),
        pltpu.VMEM((2 * T + NR,), jnp.int32),
        pltpu.VMEM((2 * T + NR,), jnp.int32),
        pltpu.VMEM((T, 2 * D), jnp.float32),
        pltpu.VMEM((T, 2 * D), jnp.float32),
        pltpu.VMEM((NR, 2 * D), jnp.float32),
        pltpu.VMEM((T, 2 * D), jnp.float32),
        pltpu.VMEM((T, 2 * D), jnp.float32),
        pltpu.VMEM((NR, 2 * D), jnp.float32),
        pltpu.VMEM((TPW,), jnp.float32),
        pltpu.VMEM((NID,), jnp.float32),
        pltpu.SemaphoreType.DMA,
        pltpu.SemaphoreType.DMA,
    ],
)(_sc_body)


def kernel(center, context, negatives, center_weight, context_weight):
    negflat = negatives.reshape(-1).astype(jnp.int32)
    pos, negf = _sc_kernel(
        center.astype(jnp.int32),
        context.astype(jnp.int32),
        negflat,
        _pair_transpose(center_weight.T),
        _pair_transpose(context_weight.T),
    )
    return pos, negf.reshape(B, NNEG)


# speedup vs baseline: 3.5241x; 1.2657x over previous
import functools

import jax
import jax.numpy as jnp
from jax import lax
from jax.experimental import pallas as pl
from jax.experimental.pallas import tpu as pltpu
from jax.experimental.pallas import tpu_sc as plsc

B = 16384
V = 1000000
TB = 8192
TBL = TB.bit_length() - 1
NBLK = -(-V // TB)
VROWS = NBLK * (TB // 2)
D = 64
NNEG = 20
L = 16
DC = D // L
NC = 2
NS = 16
NW = NC * NS
TPW = B // NW
T = 16
NCHUNK = TPW // T
NC2 = NCHUNK // 2
NR = T * NNEG
NG = NR // L
NID = TPW * NNEG
SG = 80

_DNUMS = lax.GatherDimensionNumbers(
    offset_dims=(), collapsed_slice_dims=(0,), start_index_map=(0,))


def _take(v, idx):
    return lax.gather(v, idx[:, None], _DNUMS, (1,),
                      mode=lax.GatherScatterMode.PROMISE_IN_BOUNDS)


def _bcast(v, i):
    return _take(v, jnp.full((L,), i, jnp.int32))


def _pair_body(in_ref, out_ref):
    blk = in_ref[...]
    c = jnp.concatenate([blk[:, :TB // 2], blk[:, TB // 2:]], axis=0)
    out_ref[...] = c.T


_pair_transpose = pl.pallas_call(
    _pair_body,
    grid=(NBLK,),
    in_specs=[pl.BlockSpec((D, TB), lambda i: (0, i))],
    out_specs=pl.BlockSpec((TB // 2, 2 * D), lambda i: (i, 0)),
    out_shape=jax.ShapeDtypeStruct((VROWS, 2 * D), jnp.float32),
)


class _TreeSum:
    def __init__(self, perms, masks):
        self.perms, self.masks = perms, masks
        self.stack = []

    def push(self, vec):
        lvl = 0
        while self.stack and self.stack[-1][0] == lvl:
            _, a = self.stack.pop()
            ta = a + _take(a, self.perms[lvl])
            tb = vec + _take(vec, self.perms[lvl])
            vec = jnp.where(self.masks[lvl], ta, tb)
            lvl += 1
        self.stack.append((lvl, vec))

    def finish(self):
        (_, v), = self.stack
        self.stack = []
        return v


def _sc_body(center_hbm, context_hbm, neg_hbm, cw_hbm, xw_hbm,
             pos_hbm, negout_hbm,
             cidx, xidx, nidx, hidxA, hidxB,
             crowA, xrowA, nrowA, crowB, xrowB, nrowB,
             posb, negb, semA, semB):
    cid = lax.axis_index("c")
    sid = lax.axis_index("s")
    wid = sid * NC + cid
    base = wid * TPW

    bufsA = (crowA, xrowA, nrowA)
    bufsB = (crowB, xrowB, nrowB)

    lane = lax.iota(jnp.int32, L)
    perms = [lane ^ s for s in (1, 2, 4, 8)]
    masks = [(lane & s) == 0 for s in (1, 2, 4, 8)]

    pltpu.sync_copy(center_hbm.at[pl.ds(base, TPW)], cidx)
    pltpu.sync_copy(context_hbm.at[pl.ds(base, TPW)], xidx)
    pltpu.sync_copy(neg_hbm.at[pl.ds(base * NNEG, NID)], nidx)

    def issue(c, bufs, hidx, sem):
        crow, xrow, nrow = bufs
        o = c * T
        no = c * NR
        def _pairrow(vv):
            return lax.shift_left(
                lax.shift_right_logical(vv, TBL),
                TBL - 1) | (vv & (TB // 2 - 1))

        hidx[pl.ds(0, T)] = _pairrow(cidx[pl.ds(o, T)])
        hidx[pl.ds(T, T)] = _pairrow(xidx[pl.ds(o, T)])
        for j in range(NR // L):
            hidx[pl.ds(2 * T + j * L, L)] = _pairrow(nidx[pl.ds(no + j * L, L)])
        pltpu.async_copy(cw_hbm.at[hidx.at[pl.ds(0, T)]], crow, sem)
        pltpu.async_copy(xw_hbm.at[hidx.at[pl.ds(T, T)]], xrow, sem)
        for k in range(4):
            pltpu.async_copy(xw_hbm.at[hidx.at[pl.ds(2 * T + SG * k, SG)]],
                             nrow.at[pl.ds(SG * k, SG)], sem)

    def drain(bufs, sem):
        crow, xrow, nrow = bufs
        for d, n in ((crow, T), (xrow, T), (nrow, SG), (nrow, SG),
                     (nrow, SG), (nrow, SG)):
            pltpu.make_async_copy(cw_hbm.at[pl.ds(0, n)],
                                  d.at[pl.ds(0, n)], sem).wait()

    def _halfrow(ref, r, pm):
        return [jnp.where(pm, ref[r, pl.ds(D + dc * L, L)],
                          ref[r, pl.ds(dc * L, L)]) for dc in range(DC)]

    def compute(c, bufs):
        crow, xrow, nrow = bufs
        cparv = lax.shift_right_logical(cidx[pl.ds(c * T, T)], TBL - 1) & 1
        xparv = lax.shift_right_logical(xidx[pl.ds(c * T, T)], TBL - 1) & 1
        nbase = c * NR

        def sg(sgi, carry):
            t4 = sgi * 4
            rb = sgi * SG
            for gg in range(SG // L):
                r0 = gg * L
                nparv = lax.shift_right_logical(
                    nidx[pl.ds(nbase + rb + r0, L)], TBL - 1) & 1
                cvec = {}
                tree = _TreeSum(perms, masks)
                for i in range(L):
                    rs = r0 + i
                    tloc = rs // NNEG
                    t = t4 + tloc
                    if tloc not in cvec:
                        cvec[tloc] = _halfrow(crow, t, _bcast(cparv, t) == 1)
                    nv = _halfrow(nrow, rb + rs, _bcast(nparv, i) == 1)
                    acc = nv[0] * cvec[tloc][0]
                    for dc in range(1, DC):
                        acc = acc + nv[dc] * cvec[tloc][dc]
                    tree.push(acc)
                negb[pl.ds(nbase + rb + r0, L)] = tree.finish()
            return carry

        lax.fori_loop(0, NR // SG, sg, 0)

        tree = _TreeSum(perms, masks)
        for t in range(T):
            cv = _halfrow(crow, t, _bcast(cparv, t) == 1)
            xv = _halfrow(xrow, t, _bcast(xparv, t) == 1)
            acc = cv[0] * xv[0]
            for dc in range(1, DC):
                acc = acc + cv[dc] * xv[dc]
            tree.push(acc)
        posb[pl.ds(c * T, T)] = tree.finish()

    issue(0, bufsA, hidxA, semA)

    def step(c2, carry):
        c0 = c2 * 2
        c1 = c0 + 1
        issue(c1, bufsB, hidxB, semB)
        drain(bufsA, semA)
        compute(c0, bufsA)
        cnext = lax.rem(c0 + 2, NCHUNK)
        issue(cnext, bufsA, hidxA, semA)
        drain(bufsB, semB)
        compute(c1, bufsB)
        return carry

    lax.fori_loop(0, NC2, step, 0)
    drain(bufsA, semA)

    pltpu.sync_copy(posb, pos_hbm.at[pl.ds(base, TPW)])
    pltpu.sync_copy(negb, negout_hbm.at[pl.ds(base * NNEG, NID)])


_sc_kernel = functools.partial(
    pl.kernel,
    out_type=[
        jax.ShapeDtypeStruct((B,), jnp.float32),
        jax.ShapeDtypeStruct((B * NNEG,), jnp.float32),
    ],
    mesh=plsc.VectorSubcoreMesh(core_axis_name="c", subcore_axis_name="s"),
    compiler_params=pltpu.CompilerParams(needs_layout_passes=False),
    scratch_types=[
        pltpu.VMEM((TPW,), jnp.int32),
        pltpu.VMEM((TPW,), jnp.int32),
        pltpu.VMEM((NID,), jnp.int32),
        pltpu.VMEM((2 * T + NR,), jnp.int32),
        pltpu.VMEM((2 * T + NR,), jnp.int32),
        pltpu.VMEM((T, 2 * D), jnp.float32),
        pltpu.VMEM((T, 2 * D), jnp.float32),
        pltpu.VMEM((NR, 2 * D), jnp.float32),
        pltpu.VMEM((T, 2 * D), jnp.float32),
        pltpu.VMEM((T, 2 * D), jnp.float32),
        pltpu.VMEM((NR, 2 * D), jnp.float32),
        pltpu.VMEM((TPW,), jnp.float32),
        pltpu.VMEM((NID,), jnp.float32),
        pltpu.SemaphoreType.DMA,
        pltpu.SemaphoreType.DMA,
    ],
)(_sc_body)


def kernel(center, context, negatives, center_weight, context_weight):
    negflat = negatives.reshape(-1).astype(jnp.int32)
    pos, negf = _sc_kernel(
        center.astype(jnp.int32),
        context.astype(jnp.int32),
        negflat,
        _pair_transpose(center_weight.T),
        _pair_transpose(context_weight.T),
    )
    return pos, negf.reshape(B, NNEG)


# speedup vs baseline: 3.9131x; 1.1104x over previous
import functools

import jax
import jax.numpy as jnp
from jax import lax
from jax.experimental import pallas as pl
from jax.experimental.pallas import tpu as pltpu
from jax.experimental.pallas import tpu_sc as plsc

B = 16384
V = 1000000
TB = 16384
TBL = TB.bit_length() - 1
NBLK = -(-V // TB)
VROWS = NBLK * (TB // 2)
D = 64
NNEG = 20
L = 16
DC = D // L
NC = 2
NS = 16
NW = NC * NS
TPW = B // NW
T = 16
NCHUNK = TPW // T
NC2 = NCHUNK // 2
NR = T * NNEG
NG = NR // L
NID = TPW * NNEG
SG = 80

_DNUMS = lax.GatherDimensionNumbers(
    offset_dims=(), collapsed_slice_dims=(0,), start_index_map=(0,))


def _take(v, idx):
    return lax.gather(v, idx[:, None], _DNUMS, (1,),
                      mode=lax.GatherScatterMode.PROMISE_IN_BOUNDS)


def _bcast(v, i):
    return _take(v, jnp.full((L,), i, jnp.int32))


def _pair_body(in_ref, out_ref):
    blk = in_ref[...]
    c = jnp.concatenate([blk[:, :TB // 2], blk[:, TB // 2:]], axis=0)
    out_ref[...] = c.T


_pair_transpose = pl.pallas_call(
    _pair_body,
    grid=(NBLK,),
    in_specs=[pl.BlockSpec((D, TB), lambda i: (0, i))],
    out_specs=pl.BlockSpec((TB // 2, 2 * D), lambda i: (i, 0)),
    out_shape=jax.ShapeDtypeStruct((VROWS, 2 * D), jnp.float32),
)


class _TreeSum:
    def __init__(self, perms, masks):
        self.perms, self.masks = perms, masks
        self.stack = []

    def push(self, vec):
        lvl = 0
        while self.stack and self.stack[-1][0] == lvl:
            _, a = self.stack.pop()
            ta = a + _take(a, self.perms[lvl])
            tb = vec + _take(vec, self.perms[lvl])
            vec = jnp.where(self.masks[lvl], ta, tb)
            lvl += 1
        self.stack.append((lvl, vec))

    def finish(self):
        (_, v), = self.stack
        self.stack = []
        return v


def _sc_body(center_hbm, context_hbm, neg_hbm, cw_hbm, xw_hbm,
             pos_hbm, negout_hbm,
             cidx, xidx, nidx, hidxA, hidxB,
             crowA, xrowA, nrowA, crowB, xrowB, nrowB,
             posb, negb, semA, semB):
    cid = lax.axis_index("c")
    sid = lax.axis_index("s")
    wid = sid * NC + cid
    base = wid * TPW

    bufsA = (crowA, xrowA, nrowA)
    bufsB = (crowB, xrowB, nrowB)

    lane = lax.iota(jnp.int32, L)
    perms = [lane ^ s for s in (1, 2, 4, 8)]
    masks = [(lane & s) == 0 for s in (1, 2, 4, 8)]

    pltpu.sync_copy(center_hbm.at[pl.ds(base, TPW)], cidx)
    pltpu.sync_copy(context_hbm.at[pl.ds(base, TPW)], xidx)
    pltpu.sync_copy(neg_hbm.at[pl.ds(base * NNEG, NID)], nidx)

    def issue(c, bufs, hidx, sem):
        crow, xrow, nrow = bufs
        o = c * T
        no = c * NR
        def _pairrow(vv):
            return lax.shift_left(
                lax.shift_right_logical(vv, TBL),
                TBL - 1) | (vv & (TB // 2 - 1))

        hidx[pl.ds(0, T)] = _pairrow(cidx[pl.ds(o, T)])
        hidx[pl.ds(T, T)] = _pairrow(xidx[pl.ds(o, T)])
        for j in range(NR // L):
            hidx[pl.ds(2 * T + j * L, L)] = _pairrow(nidx[pl.ds(no + j * L, L)])
        pltpu.async_copy(cw_hbm.at[hidx.at[pl.ds(0, T)]], crow, sem)
        pltpu.async_copy(xw_hbm.at[hidx.at[pl.ds(T, T)]], xrow, sem)
        for k in range(4):
            pltpu.async_copy(xw_hbm.at[hidx.at[pl.ds(2 * T + SG * k, SG)]],
                             nrow.at[pl.ds(SG * k, SG)], sem)

    def drain(bufs, sem):
        crow, xrow, nrow = bufs
        for d, n in ((crow, T), (xrow, T), (nrow, SG), (nrow, SG),
                     (nrow, SG), (nrow, SG)):
            pltpu.make_async_copy(cw_hbm.at[pl.ds(0, n)],
                                  d.at[pl.ds(0, n)], sem).wait()

    def _halfrow(ref, r, pm):
        return [jnp.where(pm, ref[r, pl.ds(D + dc * L, L)],
                          ref[r, pl.ds(dc * L, L)]) for dc in range(DC)]

    def compute(c, bufs):
        crow, xrow, nrow = bufs
        cparv = lax.shift_right_logical(cidx[pl.ds(c * T, T)], TBL - 1) & 1
        xparv = lax.shift_right_logical(xidx[pl.ds(c * T, T)], TBL - 1) & 1
        nbase = c * NR

        def sg(sgi, carry):
            t4 = sgi * 4
            rb = sgi * SG
            for gg in range(SG // L):
                r0 = gg * L
                nparv = lax.shift_right_logical(
                    nidx[pl.ds(nbase + rb + r0, L)], TBL - 1) & 1
                cvec = {}
                tree = _TreeSum(perms, masks)
                for i in range(L):
                    rs = r0 + i
                    tloc = rs // NNEG
                    t = t4 + tloc
                    if tloc not in cvec:
                        cvec[tloc] = _halfrow(crow, t, _bcast(cparv, t) == 1)
                    nv = _halfrow(nrow, rb + rs, _bcast(nparv, i) == 1)
                    acc = nv[0] * cvec[tloc][0]
                    for dc in range(1, DC):
                        acc = acc + nv[dc] * cvec[tloc][dc]
                    tree.push(acc)
                negb[pl.ds(nbase + rb + r0, L)] = tree.finish()
            return carry

        lax.fori_loop(0, NR // SG, sg, 0)

        tree = _TreeSum(perms, masks)
        for t in range(T):
            cv = _halfrow(crow, t, _bcast(cparv, t) == 1)
            xv = _halfrow(xrow, t, _bcast(xparv, t) == 1)
            acc = cv[0] * xv[0]
            for dc in range(1, DC):
                acc = acc + cv[dc] * xv[dc]
            tree.push(acc)
        posb[pl.ds(c * T, T)] = tree.finish()

    issue(0, bufsA, hidxA, semA)

    def step(c2, carry):
        c0 = c2 * 2
        c1 = c0 + 1
        issue(c1, bufsB, hidxB, semB)
        drain(bufsA, semA)
        compute(c0, bufsA)
        cnext = lax.rem(c0 + 2, NCHUNK)
        issue(cnext, bufsA, hidxA, semA)
        drain(bufsB, semB)
        compute(c1, bufsB)
        return carry

    lax.fori_loop(0, NC2, step, 0)
    drain(bufsA, semA)

    pltpu.sync_copy(posb, pos_hbm.at[pl.ds(base, TPW)])
    pltpu.sync_copy(negb, negout_hbm.at[pl.ds(base * NNEG, NID)])


_sc_kernel = functools.partial(
    pl.kernel,
    out_type=[
        jax.ShapeDtypeStruct((B,), jnp.float32),
        jax.ShapeDtypeStruct((B * NNEG,), jnp.float32),
    ],
    mesh=plsc.VectorSubcoreMesh(core_axis_name="c", subcore_axis_name="s"),
    compiler_params=pltpu.CompilerParams(needs_layout_passes=False),
    scratch_types=[
        pltpu.VMEM((TPW,), jnp.int32),
        pltpu.VMEM((TPW,), jnp.int32),
        pltpu.VMEM((NID,), jnp.int32),
        pltpu.VMEM((2 * T + NR,), jnp.int32),
        pltpu.VMEM((2 * T + NR,), jnp.int32),
        pltpu.VMEM((T, 2 * D), jnp.float32),
        pltpu.VMEM((T, 2 * D), jnp.float32),
        pltpu.VMEM((NR, 2 * D), jnp.float32),
        pltpu.VMEM((T, 2 * D), jnp.float32),
        pltpu.VMEM((T, 2 * D), jnp.float32),
        pltpu.VMEM((NR, 2 * D), jnp.float32),
        pltpu.VMEM((TPW,), jnp.float32),
        pltpu.VMEM((NID,), jnp.float32),
        pltpu.SemaphoreType.DMA,
        pltpu.SemaphoreType.DMA,
    ],
)(_sc_body)


def kernel(center, context, negatives, center_weight, context_weight):
    negflat = negatives.reshape(-1).astype(jnp.int32)
    pos, negf = _sc_kernel(
        center.astype(jnp.int32),
        context.astype(jnp.int32),
        negflat,
        _pair_transpose(center_weight.T),
        _pair_transpose(context_weight.T),
    )
    return pos, negf.reshape(B, NNEG)


# speedup vs baseline: 3.9866x; 1.0188x over previous
import functools

import jax
import jax.numpy as jnp
from jax import lax
from jax.experimental import pallas as pl
from jax.experimental.pallas import tpu as pltpu
from jax.experimental.pallas import tpu_sc as plsc

B = 16384
V = 1000000
TB = 32768
TBL = TB.bit_length() - 1
NBLK = -(-V // TB)
VROWS = NBLK * (TB // 2)
D = 64
NNEG = 20
L = 16
DC = D // L
NC = 2
NS = 16
NW = NC * NS
TPW = B // NW
T = 16
NCHUNK = TPW // T
NC2 = NCHUNK // 2
NR = T * NNEG
NG = NR // L
NID = TPW * NNEG
SG = 80

_DNUMS = lax.GatherDimensionNumbers(
    offset_dims=(), collapsed_slice_dims=(0,), start_index_map=(0,))


def _take(v, idx):
    return lax.gather(v, idx[:, None], _DNUMS, (1,),
                      mode=lax.GatherScatterMode.PROMISE_IN_BOUNDS)


def _bcast(v, i):
    return _take(v, jnp.full((L,), i, jnp.int32))


def _pair_body(in_ref, out_ref):
    blk = in_ref[...]
    c = jnp.concatenate([blk[:, :TB // 2], blk[:, TB // 2:]], axis=0)
    out_ref[...] = c.T


_pair_transpose = pl.pallas_call(
    _pair_body,
    grid=(NBLK,),
    in_specs=[pl.BlockSpec((D, TB), lambda i: (0, i))],
    out_specs=pl.BlockSpec((TB // 2, 2 * D), lambda i: (i, 0)),
    out_shape=jax.ShapeDtypeStruct((VROWS, 2 * D), jnp.float32),
)


class _TreeSum:
    def __init__(self, perms, masks):
        self.perms, self.masks = perms, masks
        self.stack = []

    def push(self, vec):
        lvl = 0
        while self.stack and self.stack[-1][0] == lvl:
            _, a = self.stack.pop()
            ta = a + _take(a, self.perms[lvl])
            tb = vec + _take(vec, self.perms[lvl])
            vec = jnp.where(self.masks[lvl], ta, tb)
            lvl += 1
        self.stack.append((lvl, vec))

    def finish(self):
        (_, v), = self.stack
        self.stack = []
        return v


def _sc_body(center_hbm, context_hbm, neg_hbm, cw_hbm, xw_hbm,
             pos_hbm, negout_hbm,
             cidx, xidx, nidx, hidxA, hidxB,
             crowA, xrowA, nrowA, crowB, xrowB, nrowB,
             posb, negb, semA, semB):
    cid = lax.axis_index("c")
    sid = lax.axis_index("s")
    wid = sid * NC + cid
    base = wid * TPW

    bufsA = (crowA, xrowA, nrowA)
    bufsB = (crowB, xrowB, nrowB)

    lane = lax.iota(jnp.int32, L)
    perms = [lane ^ s for s in (1, 2, 4, 8)]
    masks = [(lane & s) == 0 for s in (1, 2, 4, 8)]

    pltpu.sync_copy(center_hbm.at[pl.ds(base, TPW)], cidx)
    pltpu.sync_copy(context_hbm.at[pl.ds(base, TPW)], xidx)
    pltpu.sync_copy(neg_hbm.at[pl.ds(base * NNEG, NID)], nidx)

    def issue(c, bufs, hidx, sem):
        crow, xrow, nrow = bufs
        o = c * T
        no = c * NR
        def _pairrow(vv):
            return lax.shift_left(
                lax.shift_right_logical(vv, TBL),
                TBL - 1) | (vv & (TB // 2 - 1))

        hidx[pl.ds(0, T)] = _pairrow(cidx[pl.ds(o, T)])
        hidx[pl.ds(T, T)] = _pairrow(xidx[pl.ds(o, T)])
        for j in range(NR // L):
            hidx[pl.ds(2 * T + j * L, L)] = _pairrow(nidx[pl.ds(no + j * L, L)])
        pltpu.async_copy(cw_hbm.at[hidx.at[pl.ds(0, T)]], crow, sem)
        pltpu.async_copy(xw_hbm.at[hidx.at[pl.ds(T, T)]], xrow, sem)
        for k in range(4):
            pltpu.async_copy(xw_hbm.at[hidx.at[pl.ds(2 * T + SG * k, SG)]],
                             nrow.at[pl.ds(SG * k, SG)], sem)

    def drain(bufs, sem):
        crow, xrow, nrow = bufs
        for d, n in ((crow, T), (xrow, T), (nrow, SG), (nrow, SG),
                     (nrow, SG), (nrow, SG)):
            pltpu.make_async_copy(cw_hbm.at[pl.ds(0, n)],
                                  d.at[pl.ds(0, n)], sem).wait()

    def _halfrow(ref, r, pm):
        return [jnp.where(pm, ref[r, pl.ds(D + dc * L, L)],
                          ref[r, pl.ds(dc * L, L)]) for dc in range(DC)]

    def compute(c, bufs):
        crow, xrow, nrow = bufs
        cparv = lax.shift_right_logical(cidx[pl.ds(c * T, T)], TBL - 1) & 1
        xparv = lax.shift_right_logical(xidx[pl.ds(c * T, T)], TBL - 1) & 1
        nbase = c * NR

        def sg(sgi, carry):
            t4 = sgi * 4
            rb = sgi * SG
            for gg in range(SG // L):
                r0 = gg * L
                nparv = lax.shift_right_logical(
                    nidx[pl.ds(nbase + rb + r0, L)], TBL - 1) & 1
                cvec = {}
                tree = _TreeSum(perms, masks)
                for i in range(L):
                    rs = r0 + i
                    tloc = rs // NNEG
                    t = t4 + tloc
                    if tloc not in cvec:
                        cvec[tloc] = _halfrow(crow, t, _bcast(cparv, t) == 1)
                    nv = _halfrow(nrow, rb + rs, _bcast(nparv, i) == 1)
                    acc = nv[0] * cvec[tloc][0]
                    for dc in range(1, DC):
                        acc = acc + nv[dc] * cvec[tloc][dc]
                    tree.push(acc)
                negb[pl.ds(nbase + rb + r0, L)] = tree.finish()
            return carry

        lax.fori_loop(0, NR // SG, sg, 0)

        tree = _TreeSum(perms, masks)
        for t in range(T):
            cv = _halfrow(crow, t, _bcast(cparv, t) == 1)
            xv = _halfrow(xrow, t, _bcast(xparv, t) == 1)
            acc = cv[0] * xv[0]
            for dc in range(1, DC):
                acc = acc + cv[dc] * xv[dc]
            tree.push(acc)
        posb[pl.ds(c * T, T)] = tree.finish()

    issue(0, bufsA, hidxA, semA)

    def step(c2, carry):
        c0 = c2 * 2
        c1 = c0 + 1
        issue(c1, bufsB, hidxB, semB)
        drain(bufsA, semA)
        compute(c0, bufsA)
        cnext = lax.rem(c0 + 2, NCHUNK)
        issue(cnext, bufsA, hidxA, semA)
        drain(bufsB, semB)
        compute(c1, bufsB)
        return carry

    lax.fori_loop(0, NC2, step, 0)
    drain(bufsA, semA)

    pltpu.sync_copy(posb, pos_hbm.at[pl.ds(base, TPW)])
    pltpu.sync_copy(negb, negout_hbm.at[pl.ds(base * NNEG, NID)])


_sc_kernel = functools.partial(
    pl.kernel,
    out_type=[
        jax.ShapeDtypeStruct((B,), jnp.float32),
        jax.ShapeDtypeStruct((B * NNEG,), jnp.float32),
    ],
    mesh=plsc.VectorSubcoreMesh(core_axis_name="c", subcore_axis_name="s"),
    compiler_params=pltpu.CompilerParams(needs_layout_passes=False),
    scratch_types=[
        pltpu.VMEM((TPW,), jnp.int32),
        pltpu.VMEM((TPW,), jnp.int32),
        pltpu.VMEM((NID,), jnp.int32),
        pltpu.VMEM((2 * T + NR,), jnp.int32),
        pltpu.VMEM((2 * T + NR,), jnp.int32),
        pltpu.VMEM((T, 2 * D), jnp.float32),
        pltpu.VMEM((T, 2 * D), jnp.float32),
        pltpu.VMEM((NR, 2 * D), jnp.float32),
        pltpu.VMEM((T, 2 * D), jnp.float32),
        pltpu.VMEM((T, 2 * D), jnp.float32),
        pltpu.VMEM((NR, 2 * D), jnp.float32),
        pltpu.VMEM((TPW,), jnp.float32),
        pltpu.VMEM((NID,), jnp.float32),
        pltpu.SemaphoreType.DMA,
        pltpu.SemaphoreType.DMA,
    ],
)(_sc_body)


def kernel(center, context, negatives, center_weight, context_weight):
    negflat = negatives.reshape(-1).astype(jnp.int32)
    pos, negf = _sc_kernel(
        center.astype(jnp.int32),
        context.astype(jnp.int32),
        negflat,
        _pair_transpose(center_weight.T),
        _pair_transpose(context_weight.T),
    )
    return pos, negf.reshape(B, NNEG)
